# baseline (speedup 1.0000x reference)
import functools

import jax
import jax.numpy as jnp
from jax import lax
from jax.experimental import pallas as pl
from jax.experimental.pallas import tpu as pltpu
from jax.experimental.pallas import tpu_sc as plsc

NUM_NODES = 1000000
EMB = 64
B = 16384

NC = 2
NS = 16
NW = NC * NS
B_PER_W = B // NW
LANES = 16
PAIR = 2 * EMB
NP = NUM_NODES // 2
CHUNK = 128
N_CHUNKS = B_PER_W // CHUNK
GROUPS = CHUNK // LANES


def _sc_kernel(vi_hbm, vj_hbm, a_hbm, c_hbm, out_hbm,
               idx_i, idx_j, ti, tj, ga, gc, out_v, sem_a, sem_c):
    wid = lax.axis_index("s") * NC + lax.axis_index("c")
    base = wid * B_PER_W

    pltpu.sync_copy(vi_hbm.at[pl.ds(base, B_PER_W)], idx_i)
    pltpu.sync_copy(vj_hbm.at[pl.ds(base, B_PER_W)], idx_j)

    def shift_body(i, carry):
        sl = pl.ds(i * LANES, LANES)
        ti[sl] = idx_i[sl] >> 1
        tj[sl] = idx_j[sl] >> 1
        return carry

    lax.fori_loop(0, B_PER_W // LANES, shift_body, 0)

    lane = lax.iota(jnp.int32, LANES)

    def body(ch, carry):
        csl = pl.ds(ch * CHUNK, CHUNK)
        cp_a = pltpu.async_copy(a_hbm.at[ti.at[csl]], ga, sem_a)
        cp_c = pltpu.async_copy(c_hbm.at[tj.at[csl]], gc, sem_c)
        cp_a.wait()
        cp_c.wait()
        for g in range(GROUPS):
            item = lane + g * LANES
            gsl = pl.ds(ch * CHUNK + g * LANES, LANES)
            ha = (idx_i[gsl] & 1) * EMB
            hc = (idx_j[gsl] & 1) * EMB
            acc = jnp.zeros((LANES,), jnp.float32)
            for k in range(EMB):
                va = plsc.load_gather(ga, [item, ha + k])
                vc = plsc.load_gather(gc, [item, hc + k])
                acc = acc + va * vc
            out_v[gsl] = acc
        return carry

    lax.fori_loop(0, N_CHUNKS, body, 0)

    pltpu.sync_copy(out_v, out_hbm.at[pl.ds(base, B_PER_W)])


@jax.jit
def kernel(v_i, v_j, first_emb, second_emb, context_emb):
    del first_emb
    a2 = second_emb.reshape(NP, PAIR)
    c2 = context_emb.reshape(NP, PAIR)
    mesh = plsc.VectorSubcoreMesh(core_axis_name="c", subcore_axis_name="s")
    run = pl.kernel(
        _sc_kernel,
        out_type=jax.ShapeDtypeStruct((B,), jnp.float32),
        mesh=mesh,
        scratch_types=[
            pltpu.VMEM((B_PER_W,), jnp.int32),
            pltpu.VMEM((B_PER_W,), jnp.int32),
            pltpu.VMEM((B_PER_W,), jnp.int32),
            pltpu.VMEM((B_PER_W,), jnp.int32),
            pltpu.VMEM((CHUNK, PAIR), jnp.float32),
            pltpu.VMEM((CHUNK, PAIR), jnp.float32),
            pltpu.VMEM((B_PER_W,), jnp.float32),
            pltpu.SemaphoreType.DMA,
            pltpu.SemaphoreType.DMA,
        ],
        compiler_params=pltpu.CompilerParams(needs_layout_passes=False),
    )
    return run(v_i, v_j, a2, c2)


# speedup vs baseline: 1.5621x; 1.5621x over previous
import functools

import jax
import jax.numpy as jnp
from jax import lax
from jax.experimental import pallas as pl
from jax.experimental.pallas import tpu as pltpu
from jax.experimental.pallas import tpu_sc as plsc

NUM_NODES = 1000000
EMB = 64
B = 16384

NC = 2
NS = 16
NW = NC * NS
B_PER_W = B // NW
LANES = 16
CHUNK = 128
N_CHUNKS = B_PER_W // CHUNK
GROUPS = CHUNK // LANES


def _sc_kernel(vi_hbm, vj_hbm, a_hbm, c_hbm, out_hbm,
               idx_i, idx_j, buf_a, buf_c, out_v, sem_a, sem_c):
    wid = lax.axis_index("s") * NC + lax.axis_index("c")
    base = wid * B_PER_W

    pltpu.sync_copy(vi_hbm.at[pl.ds(base, B_PER_W)], idx_i)
    pltpu.sync_copy(vj_hbm.at[pl.ds(base, B_PER_W)], idx_j)

    lane = lax.iota(jnp.int32, LANES)

    def chunk_body(ch, carry):
        def issue_body(g, icarry):
            vi = idx_i[pl.ds(ch * CHUNK + g * LANES, LANES)]
            vj = idx_j[pl.ds(ch * CHUNK + g * LANES, LANES)]
            for r in range(LANES):
                d = g * LANES + r
                pltpu.async_copy(a_hbm.at[pl.ds(vi[r], 1)],
                                 buf_a.at[pl.ds(d, 1)], sem_a)
                pltpu.async_copy(c_hbm.at[pl.ds(vj[r], 1)],
                                 buf_c.at[pl.ds(d, 1)], sem_c)
            return icarry

        lax.fori_loop(0, GROUPS, issue_body, 0)

        pltpu.make_async_copy(a_hbm.at[pl.ds(0, CHUNK)], buf_a, sem_a).wait()
        pltpu.make_async_copy(c_hbm.at[pl.ds(0, CHUNK)], buf_c, sem_c).wait()

        def compute_body(g, ccarry):
            item = lane + g * LANES
            acc = jnp.zeros((LANES,), jnp.float32)
            for k in range(EMB):
                kvec = jnp.full((LANES,), k, jnp.int32)
                va = plsc.load_gather(buf_a, [item, kvec])
                vc = plsc.load_gather(buf_c, [item, kvec])
                acc = acc + va * vc
            out_v[pl.ds(ch * CHUNK + g * LANES, LANES)] = acc
            return ccarry

        lax.fori_loop(0, GROUPS, compute_body, 0)
        return carry

    lax.fori_loop(0, N_CHUNKS, chunk_body, 0)

    pltpu.sync_copy(out_v, out_hbm.at[pl.ds(base, B_PER_W)])


@jax.jit
def kernel(v_i, v_j, first_emb, second_emb, context_emb):
    del first_emb
    mesh = plsc.VectorSubcoreMesh(core_axis_name="c", subcore_axis_name="s")
    run = pl.kernel(
        _sc_kernel,
        out_type=jax.ShapeDtypeStruct((B,), jnp.float32),
        mesh=mesh,
        scratch_types=[
            pltpu.VMEM((B_PER_W,), jnp.int32),
            pltpu.VMEM((B_PER_W,), jnp.int32),
            pltpu.VMEM((CHUNK, EMB), jnp.float32),
            pltpu.VMEM((CHUNK, EMB), jnp.float32),
            pltpu.VMEM((B_PER_W,), jnp.float32),
            pltpu.SemaphoreType.DMA,
            pltpu.SemaphoreType.DMA,
        ],
        compiler_params=pltpu.CompilerParams(needs_layout_passes=False),
    )
    return run(v_i, v_j, second_emb, context_emb)


# speedup vs baseline: 1.5660x; 1.0025x over previous
import functools

import jax
import jax.numpy as jnp
from jax import lax
from jax.experimental import pallas as pl
from jax.experimental.pallas import tpu as pltpu
from jax.experimental.pallas import tpu_sc as plsc

NUM_NODES = 1000000
EMB = 64
B = 16384

NC = 2
NS = 16
NW = NC * NS
B_PER_W = B // NW
LANES = 16
CHUNK = 128
N_CHUNKS = B_PER_W // CHUNK
GROUPS = CHUNK // LANES

TBLK = 2048


def _transpose_body(at_ref, ct_ref, a_out, c_out):
    a_out[...] = at_ref[...].T
    c_out[...] = ct_ref[...].T


def _tc_transpose(at, ct):
    n = at.shape[1]
    return pl.pallas_call(
        _transpose_body,
        grid=(pl.cdiv(n, TBLK),),
        in_specs=[pl.BlockSpec((EMB, TBLK), lambda g: (0, g)),
                  pl.BlockSpec((EMB, TBLK), lambda g: (0, g))],
        out_specs=[pl.BlockSpec((TBLK, EMB), lambda g: (g, 0)),
                   pl.BlockSpec((TBLK, EMB), lambda g: (g, 0))],
        out_shape=[jax.ShapeDtypeStruct((n, EMB), jnp.float32),
                   jax.ShapeDtypeStruct((n, EMB), jnp.float32)],
    )(at, ct)


def _sc_kernel(vi_hbm, vj_hbm, a_hbm, c_hbm, out_hbm,
               idx_i, idx_j, buf_a, buf_c, out_v, sem_a, sem_c):
    wid = lax.axis_index("s") * NC + lax.axis_index("c")
    base = wid * B_PER_W

    pltpu.sync_copy(vi_hbm.at[pl.ds(base, B_PER_W)], idx_i)
    pltpu.sync_copy(vj_hbm.at[pl.ds(base, B_PER_W)], idx_j)

    lane = lax.iota(jnp.int32, LANES)

    def chunk_body(ch, carry):
        def issue_body(g, icarry):
            vi = idx_i[pl.ds(ch * CHUNK + g * LANES, LANES)]
            vj = idx_j[pl.ds(ch * CHUNK + g * LANES, LANES)]
            for r in range(LANES):
                d = g * LANES + r
                pltpu.async_copy(a_hbm.at[pl.ds(vi[r], 1)],
                                 buf_a.at[pl.ds(d, 1)], sem_a)
                pltpu.async_copy(c_hbm.at[pl.ds(vj[r], 1)],
                                 buf_c.at[pl.ds(d, 1)], sem_c)
            return icarry

        lax.fori_loop(0, GROUPS, issue_body, 0)

        pltpu.make_async_copy(a_hbm.at[pl.ds(0, CHUNK)], buf_a, sem_a).wait()
        pltpu.make_async_copy(c_hbm.at[pl.ds(0, CHUNK)], buf_c, sem_c).wait()

        def compute_body(g, ccarry):
            item = lane + g * LANES
            acc = jnp.zeros((LANES,), jnp.float32)
            for k in range(EMB):
                kvec = jnp.full((LANES,), k, jnp.int32)
                va = plsc.load_gather(buf_a, [item, kvec])
                vc = plsc.load_gather(buf_c, [item, kvec])
                acc = acc + va * vc
            out_v[pl.ds(ch * CHUNK + g * LANES, LANES)] = acc
            return ccarry

        lax.fori_loop(0, GROUPS, compute_body, 0)
        return carry

    lax.fori_loop(0, N_CHUNKS, chunk_body, 0)

    pltpu.sync_copy(out_v, out_hbm.at[pl.ds(base, B_PER_W)])


@jax.jit
def kernel(v_i, v_j, first_emb, second_emb, context_emb):
    del first_emb
    a_rm, c_rm = _tc_transpose(second_emb.T, context_emb.T)
    mesh = plsc.VectorSubcoreMesh(core_axis_name="c", subcore_axis_name="s")
    run = pl.kernel(
        _sc_kernel,
        out_type=jax.ShapeDtypeStruct((B,), jnp.float32),
        mesh=mesh,
        scratch_types=[
            pltpu.VMEM((B_PER_W,), jnp.int32),
            pltpu.VMEM((B_PER_W,), jnp.int32),
            pltpu.VMEM((CHUNK, EMB), jnp.float32),
            pltpu.VMEM((CHUNK, EMB), jnp.float32),
            pltpu.VMEM((B_PER_W,), jnp.float32),
            pltpu.SemaphoreType.DMA,
            pltpu.SemaphoreType.DMA,
        ],
        compiler_params=pltpu.CompilerParams(needs_layout_passes=False),
    )
    return run(v_i, v_j, a_rm, c_rm)


# speedup vs baseline: 1.6215x; 1.0355x over previous
import functools

import jax
import jax.numpy as jnp
from jax import lax
from jax.experimental import pallas as pl
from jax.experimental.pallas import tpu as pltpu
from jax.experimental.pallas import tpu_sc as plsc

NUM_NODES = 1000000
EMB = 64
B = 16384

NC = 2
NS = 16
NW = NC * NS
B_PER_W = B // NW
LANES = 16
PAIR = 2 * EMB
HALF = 500736
CHUNK = 128
N_CHUNKS = B_PER_W // CHUNK
GROUPS = CHUNK // LANES

TBLK = 1024


def _transpose_body(a1, a2, c1, c2, a_out, c_out):
    a_out[...] = jnp.concatenate([a1[...].T, a2[...].T], axis=1)
    c_out[...] = jnp.concatenate([c1[...].T, c2[...].T], axis=1)


def _tc_transpose(at, ct):
    grid = (HALF // TBLK,)
    last_in_blk = (NUM_NODES + TBLK - 1) // TBLK - 1
    lo = lambda g: (0, g)
    hi = lambda g: (0, jnp.minimum(g + grid[0], last_in_blk))
    return pl.pallas_call(
        _transpose_body,
        grid=grid,
        in_specs=[pl.BlockSpec((EMB, TBLK), lo), pl.BlockSpec((EMB, TBLK), hi),
                  pl.BlockSpec((EMB, TBLK), lo), pl.BlockSpec((EMB, TBLK), hi)],
        out_specs=[pl.BlockSpec((TBLK, PAIR), lambda g: (g, 0)),
                   pl.BlockSpec((TBLK, PAIR), lambda g: (g, 0))],
        out_shape=[jax.ShapeDtypeStruct((HALF, PAIR), jnp.float32),
                   jax.ShapeDtypeStruct((HALF, PAIR), jnp.float32)],
    )(at, at, ct, ct)


def _sc_kernel(vi_hbm, vj_hbm, a_hbm, c_hbm, out_hbm,
               idx_i, idx_j, ti, tj, ga, gc, out_v, sem_a, sem_c):
    wid = lax.axis_index("s") * NC + lax.axis_index("c")
    base = wid * B_PER_W

    pltpu.sync_copy(vi_hbm.at[pl.ds(base, B_PER_W)], idx_i)
    pltpu.sync_copy(vj_hbm.at[pl.ds(base, B_PER_W)], idx_j)

    def shift_body(i, carry):
        sl = pl.ds(i * LANES, LANES)
        vi = idx_i[sl]
        vj = idx_j[sl]
        ti[sl] = jnp.where(vi >= HALF, vi - HALF, vi)
        tj[sl] = jnp.where(vj >= HALF, vj - HALF, vj)
        return carry

    lax.fori_loop(0, B_PER_W // LANES, shift_body, 0)

    lane = lax.iota(jnp.int32, LANES)

    def body(ch, carry):
        csl = pl.ds(ch * CHUNK, CHUNK)
        cp_a = pltpu.async_copy(a_hbm.at[ti.at[csl]], ga, sem_a)
        cp_c = pltpu.async_copy(c_hbm.at[tj.at[csl]], gc, sem_c)
        cp_a.wait()
        cp_c.wait()
        for g in range(GROUPS):
            item = lane + g * LANES
            gsl = pl.ds(ch * CHUNK + g * LANES, LANES)
            vi = idx_i[gsl]
            vj = idx_j[gsl]
            ha = jnp.where(vi >= HALF, EMB, 0)
            hc = jnp.where(vj >= HALF, EMB, 0)
            acc = jnp.zeros((LANES,), jnp.float32)
            for k in range(EMB):
                va = plsc.load_gather(ga, [item, ha + k])
                vc = plsc.load_gather(gc, [item, hc + k])
                acc = acc + va * vc
            out_v[gsl] = acc
        return carry

    lax.fori_loop(0, N_CHUNKS, body, 0)

    pltpu.sync_copy(out_v, out_hbm.at[pl.ds(base, B_PER_W)])


@jax.jit
def kernel(v_i, v_j, first_emb, second_emb, context_emb):
    del first_emb
    a_rm, c_rm = _tc_transpose(second_emb.T, context_emb.T)
    mesh = plsc.VectorSubcoreMesh(core_axis_name="c", subcore_axis_name="s")
    run = pl.kernel(
        _sc_kernel,
        out_type=jax.ShapeDtypeStruct((B,), jnp.float32),
        mesh=mesh,
        scratch_types=[
            pltpu.VMEM((B_PER_W,), jnp.int32),
            pltpu.VMEM((B_PER_W,), jnp.int32),
            pltpu.VMEM((B_PER_W,), jnp.int32),
            pltpu.VMEM((B_PER_W,), jnp.int32),
            pltpu.VMEM((CHUNK, PAIR), jnp.float32),
            pltpu.VMEM((CHUNK, PAIR), jnp.float32),
            pltpu.VMEM((B_PER_W,), jnp.float32),
            pltpu.SemaphoreType.DMA,
            pltpu.SemaphoreType.DMA,
        ],
        compiler_params=pltpu.CompilerParams(needs_layout_passes=False),
    )
    return run(v_i, v_j, a_rm, c_rm)


# speedup vs baseline: 2.3848x; 1.4707x over previous
import functools

import jax
import jax.numpy as jnp
from jax import lax
from jax.experimental import pallas as pl
from jax.experimental.pallas import tpu as pltpu
from jax.experimental.pallas import tpu_sc as plsc

NUM_NODES = 1000000
EMB = 64
B = 16384

NC = 2
NS = 16
NW = NC * NS
B_PER_W = B // NW
LANES = 16
PAIR = 2 * EMB
HALF = 503808
CHUNK = 128
N_CHUNKS = B_PER_W // CHUNK
GROUPS = CHUNK // LANES

TBLK = 4096


def _transpose_body(a1, a2, c1, c2, a_out, c_out):
    eye = jnp.eye(EMB, dtype=jnp.float32)

    def tp(ref):
        return jax.lax.dot_general(ref[...], eye, (((0,), (0,)), ((), ())),
                                   preferred_element_type=jnp.float32)

    a_out[...] = jnp.concatenate([tp(a1), tp(a2)], axis=1)
    c_out[...] = jnp.concatenate([tp(c1), tp(c2)], axis=1)


def _tc_transpose(at, ct):
    grid = (HALF // TBLK,)
    last_in_blk = (NUM_NODES + TBLK - 1) // TBLK - 1
    lo = lambda g: (0, g)
    hi = lambda g: (0, jnp.minimum(g + grid[0], last_in_blk))
    return pl.pallas_call(
        _transpose_body,
        grid=grid,
        in_specs=[pl.BlockSpec((EMB, TBLK), lo), pl.BlockSpec((EMB, TBLK), hi),
                  pl.BlockSpec((EMB, TBLK), lo), pl.BlockSpec((EMB, TBLK), hi)],
        out_specs=[pl.BlockSpec((TBLK, PAIR), lambda g: (g, 0)),
                   pl.BlockSpec((TBLK, PAIR), lambda g: (g, 0))],
        out_shape=[jax.ShapeDtypeStruct((HALF, PAIR), jnp.float32),
                   jax.ShapeDtypeStruct((HALF, PAIR), jnp.float32)],
    )(at, at, ct, ct)


def _sc_kernel(vi_hbm, vj_hbm, a_hbm, c_hbm, out_hbm,
               idx_i, idx_j, ti, tj, ga, gc, out_v, sem_a, sem_c):
    wid = lax.axis_index("s") * NC + lax.axis_index("c")
    base = wid * B_PER_W

    pltpu.sync_copy(vi_hbm.at[pl.ds(base, B_PER_W)], idx_i)
    pltpu.sync_copy(vj_hbm.at[pl.ds(base, B_PER_W)], idx_j)

    def shift_body(i, carry):
        sl = pl.ds(i * LANES, LANES)
        vi = idx_i[sl]
        vj = idx_j[sl]
        ti[sl] = jnp.where(vi >= HALF, vi - HALF, vi)
        tj[sl] = jnp.where(vj >= HALF, vj - HALF, vj)
        return carry

    lax.fori_loop(0, B_PER_W // LANES, shift_body, 0)

    lane = lax.iota(jnp.int32, LANES)

    def body(ch, carry):
        csl = pl.ds(ch * CHUNK, CHUNK)
        cp_a = pltpu.async_copy(a_hbm.at[ti.at[csl]], ga, sem_a)
        cp_c = pltpu.async_copy(c_hbm.at[tj.at[csl]], gc, sem_c)
        cp_a.wait()
        cp_c.wait()
        for g in range(GROUPS):
            item = lane + g * LANES
            gsl = pl.ds(ch * CHUNK + g * LANES, LANES)
            vi = idx_i[gsl]
            vj = idx_j[gsl]
            ha = jnp.where(vi >= HALF, EMB, 0)
            hc = jnp.where(vj >= HALF, EMB, 0)
            acc = jnp.zeros((LANES,), jnp.float32)
            for k in range(EMB):
                va = plsc.load_gather(ga, [item, ha + k])
                vc = plsc.load_gather(gc, [item, hc + k])
                acc = acc + va * vc
            out_v[gsl] = acc
        return carry

    lax.fori_loop(0, N_CHUNKS, body, 0)

    pltpu.sync_copy(out_v, out_hbm.at[pl.ds(base, B_PER_W)])


@jax.jit
def kernel(v_i, v_j, first_emb, second_emb, context_emb):
    del first_emb
    a_rm, c_rm = _tc_transpose(second_emb.T, context_emb.T)
    mesh = plsc.VectorSubcoreMesh(core_axis_name="c", subcore_axis_name="s")
    run = pl.kernel(
        _sc_kernel,
        out_type=jax.ShapeDtypeStruct((B,), jnp.float32),
        mesh=mesh,
        scratch_types=[
            pltpu.VMEM((B_PER_W,), jnp.int32),
            pltpu.VMEM((B_PER_W,), jnp.int32),
            pltpu.VMEM((B_PER_W,), jnp.int32),
            pltpu.VMEM((B_PER_W,), jnp.int32),
            pltpu.VMEM((CHUNK, PAIR), jnp.float32),
            pltpu.VMEM((CHUNK, PAIR), jnp.float32),
            pltpu.VMEM((B_PER_W,), jnp.float32),
            pltpu.SemaphoreType.DMA,
            pltpu.SemaphoreType.DMA,
        ],
        compiler_params=pltpu.CompilerParams(needs_layout_passes=False),
    )
    return run(v_i, v_j, a_rm, c_rm)


# speedup vs baseline: 2.4201x; 1.0148x over previous
import functools

import jax
import jax.numpy as jnp
from jax import lax
from jax.experimental import pallas as pl
from jax.experimental.pallas import tpu as pltpu
from jax.experimental.pallas import tpu_sc as plsc

NUM_NODES = 1000000
EMB = 64
B = 16384

NC = 2
NS = 16
NW = NC * NS
B_PER_W = B // NW
LANES = 16
PAIR = 2 * EMB
HALF = 507904
CHUNK = 128
N_CHUNKS = B_PER_W // CHUNK
GROUPS = CHUNK // LANES

TBLK = 8192


def _transpose_body(a1, a2, c1, c2, a_out, c_out):
    eye = jnp.eye(EMB, dtype=jnp.float32)

    def tp(ref):
        return jax.lax.dot_general(ref[...], eye, (((0,), (0,)), ((), ())),
                                   preferred_element_type=jnp.float32)

    a_out[...] = jnp.concatenate([tp(a1), tp(a2)], axis=1)
    c_out[...] = jnp.concatenate([tp(c1), tp(c2)], axis=1)


def _tc_transpose(at, ct):
    grid = (HALF // TBLK,)
    last_in_blk = (NUM_NODES + TBLK - 1) // TBLK - 1
    lo = lambda g: (0, g)
    hi = lambda g: (0, jnp.minimum(g + grid[0], last_in_blk))
    return pl.pallas_call(
        _transpose_body,
        grid=grid,
        in_specs=[pl.BlockSpec((EMB, TBLK), lo), pl.BlockSpec((EMB, TBLK), hi),
                  pl.BlockSpec((EMB, TBLK), lo), pl.BlockSpec((EMB, TBLK), hi)],
        out_specs=[pl.BlockSpec((TBLK, PAIR), lambda g: (g, 0)),
                   pl.BlockSpec((TBLK, PAIR), lambda g: (g, 0))],
        out_shape=[jax.ShapeDtypeStruct((HALF, PAIR), jnp.float32),
                   jax.ShapeDtypeStruct((HALF, PAIR), jnp.float32)],
    )(at, at, ct, ct)


def _sc_kernel(vi_hbm, vj_hbm, a_hbm, c_hbm, out_hbm,
               idx_i, idx_j, ti, tj, ga, gc, out_v, sem_a, sem_c):
    wid = lax.axis_index("s") * NC + lax.axis_index("c")
    base = wid * B_PER_W

    pltpu.sync_copy(vi_hbm.at[pl.ds(base, B_PER_W)], idx_i)
    pltpu.sync_copy(vj_hbm.at[pl.ds(base, B_PER_W)], idx_j)

    def shift_body(i, carry):
        sl = pl.ds(i * LANES, LANES)
        vi = idx_i[sl]
        vj = idx_j[sl]
        ti[sl] = jnp.where(vi >= HALF, vi - HALF, vi)
        tj[sl] = jnp.where(vj >= HALF, vj - HALF, vj)
        return carry

    lax.fori_loop(0, B_PER_W // LANES, shift_body, 0)

    lane = lax.iota(jnp.int32, LANES)

    def body(ch, carry):
        csl = pl.ds(ch * CHUNK, CHUNK)
        cp_a = pltpu.async_copy(a_hbm.at[ti.at[csl]], ga, sem_a)
        cp_c = pltpu.async_copy(c_hbm.at[tj.at[csl]], gc, sem_c)
        cp_a.wait()
        cp_c.wait()
        for g in range(GROUPS):
            item = lane + g * LANES
            gsl = pl.ds(ch * CHUNK + g * LANES, LANES)
            vi = idx_i[gsl]
            vj = idx_j[gsl]
            ha = jnp.where(vi >= HALF, EMB, 0)
            hc = jnp.where(vj >= HALF, EMB, 0)
            acc = jnp.zeros((LANES,), jnp.float32)
            for k in range(EMB):
                va = plsc.load_gather(ga, [item, ha + k])
                vc = plsc.load_gather(gc, [item, hc + k])
                acc = acc + va * vc
            out_v[gsl] = acc
        return carry

    lax.fori_loop(0, N_CHUNKS, body, 0)

    pltpu.sync_copy(out_v, out_hbm.at[pl.ds(base, B_PER_W)])


@jax.jit
def kernel(v_i, v_j, first_emb, second_emb, context_emb):
    del first_emb
    a_rm, c_rm = _tc_transpose(second_emb.T, context_emb.T)
    mesh = plsc.VectorSubcoreMesh(core_axis_name="c", subcore_axis_name="s")
    run = pl.kernel(
        _sc_kernel,
        out_type=jax.ShapeDtypeStruct((B,), jnp.float32),
        mesh=mesh,
        scratch_types=[
            pltpu.VMEM((B_PER_W,), jnp.int32),
            pltpu.VMEM((B_PER_W,), jnp.int32),
            pltpu.VMEM((B_PER_W,), jnp.int32),
            pltpu.VMEM((B_PER_W,), jnp.int32),
            pltpu.VMEM((CHUNK, PAIR), jnp.float32),
            pltpu.VMEM((CHUNK, PAIR), jnp.float32),
            pltpu.VMEM((B_PER_W,), jnp.float32),
            pltpu.SemaphoreType.DMA,
            pltpu.SemaphoreType.DMA,
        ],
        compiler_params=pltpu.CompilerParams(needs_layout_passes=False),
    )
    return run(v_i, v_j, a_rm, c_rm)


# speedup vs baseline: 2.4249x; 1.0020x over previous
import functools

import jax
import jax.numpy as jnp
from jax import lax
from jax.experimental import pallas as pl
from jax.experimental.pallas import tpu as pltpu
from jax.experimental.pallas import tpu_sc as plsc

NUM_NODES = 1000000
EMB = 64
B = 16384

NC = 2
NS = 16
NW = NC * NS
B_PER_W = B // NW
LANES = 16
PAIR = 2 * EMB
HALF = 507904
CHUNK = 128
N_CHUNKS = B_PER_W // CHUNK
GROUPS = CHUNK // LANES

TBLK = 8192


def _transpose_body(a1, a2, c1, c2, a_out, c_out):
    a_out[...] = jnp.concatenate([a1[...].T, a2[...].T], axis=1)
    c_out[...] = jnp.concatenate([c1[...].T, c2[...].T], axis=1)


def _tc_transpose(at, ct):
    grid = (HALF // TBLK,)
    last_in_blk = (NUM_NODES + TBLK - 1) // TBLK - 1
    lo = lambda g: (0, g)
    hi = lambda g: (0, jnp.minimum(g + grid[0], last_in_blk))
    return pl.pallas_call(
        _transpose_body,
        grid=grid,
        in_specs=[pl.BlockSpec((EMB, TBLK), lo), pl.BlockSpec((EMB, TBLK), hi),
                  pl.BlockSpec((EMB, TBLK), lo), pl.BlockSpec((EMB, TBLK), hi)],
        out_specs=[pl.BlockSpec((TBLK, PAIR), lambda g: (g, 0)),
                   pl.BlockSpec((TBLK, PAIR), lambda g: (g, 0))],
        out_shape=[jax.ShapeDtypeStruct((HALF, PAIR), jnp.float32),
                   jax.ShapeDtypeStruct((HALF, PAIR), jnp.float32)],
    )(at, at, ct, ct)


def _sc_kernel(vi_hbm, vj_hbm, a_hbm, c_hbm, out_hbm,
               idx_i, idx_j, ti, tj, ga, gc, out_v, sem_a, sem_c):
    wid = lax.axis_index("s") * NC + lax.axis_index("c")
    base = wid * B_PER_W

    pltpu.sync_copy(vi_hbm.at[pl.ds(base, B_PER_W)], idx_i)
    pltpu.sync_copy(vj_hbm.at[pl.ds(base, B_PER_W)], idx_j)

    def shift_body(i, carry):
        sl = pl.ds(i * LANES, LANES)
        vi = idx_i[sl]
        vj = idx_j[sl]
        ti[sl] = jnp.where(vi >= HALF, vi - HALF, vi)
        tj[sl] = jnp.where(vj >= HALF, vj - HALF, vj)
        return carry

    lax.fori_loop(0, B_PER_W // LANES, shift_body, 0)

    lane = lax.iota(jnp.int32, LANES)

    def body(ch, carry):
        csl = pl.ds(ch * CHUNK, CHUNK)
        cp_a = pltpu.async_copy(a_hbm.at[ti.at[csl]], ga, sem_a)
        cp_c = pltpu.async_copy(c_hbm.at[tj.at[csl]], gc, sem_c)
        cp_a.wait()
        cp_c.wait()
        for g in range(GROUPS):
            item = lane + g * LANES
            gsl = pl.ds(ch * CHUNK + g * LANES, LANES)
            vi = idx_i[gsl]
            vj = idx_j[gsl]
            ha = jnp.where(vi >= HALF, EMB, 0)
            hc = jnp.where(vj >= HALF, EMB, 0)
            acc = jnp.zeros((LANES,), jnp.float32)
            for k in range(EMB):
                va = plsc.load_gather(ga, [item, ha + k])
                vc = plsc.load_gather(gc, [item, hc + k])
                acc = acc + va * vc
            out_v[gsl] = acc
        return carry

    lax.fori_loop(0, N_CHUNKS, body, 0)

    pltpu.sync_copy(out_v, out_hbm.at[pl.ds(base, B_PER_W)])


@jax.jit
def kernel(v_i, v_j, first_emb, second_emb, context_emb):
    del first_emb
    a_rm, c_rm = _tc_transpose(second_emb.T, context_emb.T)
    mesh = plsc.VectorSubcoreMesh(core_axis_name="c", subcore_axis_name="s")
    run = pl.kernel(
        _sc_kernel,
        out_type=jax.ShapeDtypeStruct((B,), jnp.float32),
        mesh=mesh,
        scratch_types=[
            pltpu.VMEM((B_PER_W,), jnp.int32),
            pltpu.VMEM((B_PER_W,), jnp.int32),
            pltpu.VMEM((B_PER_W,), jnp.int32),
            pltpu.VMEM((B_PER_W,), jnp.int32),
            pltpu.VMEM((CHUNK, PAIR), jnp.float32),
            pltpu.VMEM((CHUNK, PAIR), jnp.float32),
            pltpu.VMEM((B_PER_W,), jnp.float32),
            pltpu.SemaphoreType.DMA,
            pltpu.SemaphoreType.DMA,
        ],
        compiler_params=pltpu.CompilerParams(needs_layout_passes=False),
    )
    return run(v_i, v_j, a_rm, c_rm)
